# baseline (speedup 1.0000x reference)
import functools

import jax
import jax.numpy as jnp
from jax.experimental import pallas as pl
from jax.experimental.pallas import tpu as pltpu


def _band_kernel(cur_ref, up_ref, down_ref, w_ref, b_ref, out_ref, *, W, HB):
    NB = HB * W
    cur = cur_ref[0]
    up_row = up_ref[0, :, 0, 0, :]
    down_row = down_ref[0, :, 0, 0, :]

    up = jnp.concatenate([up_row, cur[:, : NB - W]], axis=1)
    down = jnp.concatenate([cur[:, W:], down_row], axis=1)
    colsum = up + cur + down

    wpos = jax.lax.broadcasted_iota(jnp.int32, (1, NB), 1) % W
    left = jnp.concatenate([colsum[:, :1], colsum[:, :-1]], axis=1)
    left = jnp.where(wpos == 0, colsum, left)
    right = jnp.concatenate([colsum[:, 1:], colsum[:, -1:]], axis=1)
    right = jnp.where(wpos == W - 1, colsum, right)
    sum9 = left + colsum + right

    mean = (sum9 - cur) * 0.125

    agg = jnp.concatenate([cur, mean], axis=0)
    y = jnp.dot(w_ref[...], agg, preferred_element_type=jnp.float32)
    y = y + b_ref[...]
    out_ref[0] = jnp.maximum(y, 0.0) + cur


def kernel(x, W_proj, b_proj):
    B, C, H, W = x.shape
    N = H * W
    HB = 56
    nbands = H // HB
    NB = HB * W

    x2 = x.reshape(B, C, N)
    x5 = x.reshape(B, C, H, 1, W)
    b2 = b_proj.reshape(C, 1)

    grid = (B, nbands)
    out = pl.pallas_call(
        functools.partial(_band_kernel, W=W, HB=HB),
        grid=grid,
        in_specs=[
            pl.BlockSpec((1, C, NB), lambda b, h: (b, 0, h)),
            pl.BlockSpec(
                (1, C, 1, 1, W),
                lambda b, h: (b, 0, jnp.maximum(h * HB - 1, 0), 0, 0),
            ),
            pl.BlockSpec(
                (1, C, 1, 1, W),
                lambda b, h: (b, 0, jnp.minimum((h + 1) * HB, H - 1), 0, 0),
            ),
            pl.BlockSpec((C, 2 * C), lambda b, h: (0, 0)),
            pl.BlockSpec((C, 1), lambda b, h: (0, 0)),
        ],
        out_specs=pl.BlockSpec((1, C, NB), lambda b, h: (b, 0, h)),
        out_shape=jax.ShapeDtypeStruct((B, C, N), jnp.float32),
        compiler_params=pltpu.CompilerParams(
            dimension_semantics=("parallel", "arbitrary"),
        ),
    )(x2, x5, x5, W_proj, b2)
    return out.reshape(B, C, H, W)


# speedup vs baseline: 1.7212x; 1.7212x over previous
import functools

import jax
import jax.numpy as jnp
from jax.experimental import pallas as pl
from jax.experimental.pallas import tpu as pltpu


def _band_kernel(cur_ref, up_ref, down_ref, w_ref, b_ref, out_ref, *, W, HB):
    NB = HB * W
    h = pl.program_id(1)
    nbands = pl.num_programs(1)
    cur = cur_ref[0]

    up_blk = up_ref[0]
    down_blk = down_ref[0]
    up_row = jnp.where(h == 0, up_blk[:, :W], up_blk[:, 3 * W:])
    down_row = jnp.where(h == nbands - 1, down_blk[:, 3 * W:], down_blk[:, :W])

    up = jnp.concatenate([up_row, cur[:, : NB - W]], axis=1)
    down = jnp.concatenate([cur[:, W:], down_row], axis=1)
    colsum = up + cur + down

    wpos = jax.lax.broadcasted_iota(jnp.int32, (1, NB), 1) % W
    left = jnp.concatenate([colsum[:, :1], colsum[:, :-1]], axis=1)
    left = jnp.where(wpos == 0, colsum, left)
    right = jnp.concatenate([colsum[:, 1:], colsum[:, -1:]], axis=1)
    right = jnp.where(wpos == W - 1, colsum, right)
    sum9 = left + colsum + right

    mean = (sum9 - cur) * 0.125

    agg = jnp.concatenate([cur, mean], axis=0)
    y = jnp.dot(w_ref[...], agg, preferred_element_type=jnp.float32)
    y = y + b_ref[...]
    out_ref[0] = jnp.maximum(y, 0.0) + cur


def kernel(x, W_proj, b_proj):
    B, C, H, W = x.shape
    N = H * W
    HB = 56
    nbands = H // HB
    NB = HB * W
    RPB = HB // 4

    x2 = x.reshape(B, C, N)
    b2 = b_proj.reshape(C, 1)

    grid = (B, nbands)
    out = pl.pallas_call(
        functools.partial(_band_kernel, W=W, HB=HB),
        grid=grid,
        in_specs=[
            pl.BlockSpec((1, C, NB), lambda b, h: (b, 0, h)),
            pl.BlockSpec(
                (1, C, 4 * W),
                lambda b, h: (b, 0, jnp.maximum(h * RPB - 1, 0)),
            ),
            pl.BlockSpec(
                (1, C, 4 * W),
                lambda b, h: (b, 0, jnp.minimum((h + 1) * RPB, nbands * RPB - 1)),
            ),
            pl.BlockSpec((C, 2 * C), lambda b, h: (0, 0)),
            pl.BlockSpec((C, 1), lambda b, h: (0, 0)),
        ],
        out_specs=pl.BlockSpec((1, C, NB), lambda b, h: (b, 0, h)),
        out_shape=jax.ShapeDtypeStruct((B, C, N), jnp.float32),
        compiler_params=pltpu.CompilerParams(
            dimension_semantics=("parallel", "arbitrary"),
        ),
    )(x2, x2, x2, W_proj, b2)
    return out.reshape(B, C, H, W)


# speedup vs baseline: 1.7370x; 1.0092x over previous
import functools

import jax
import jax.numpy as jnp
from jax.experimental import pallas as pl
from jax.experimental.pallas import tpu as pltpu


def _band_kernel(cur_ref, up_ref, down_ref, w_ref, b_ref, out_ref, *, W, HB):
    NB = HB * W
    h = pl.program_id(1)
    nbands = pl.num_programs(1)
    cur = cur_ref[0]

    up_blk = up_ref[0]
    down_blk = down_ref[0]
    up_row = jnp.where(h == 0, up_blk[:, :W], up_blk[:, 3 * W:])
    down_row = jnp.where(h == nbands - 1, down_blk[:, 3 * W:], down_blk[:, :W])

    up = jnp.concatenate([up_row, cur[:, : NB - W]], axis=1)
    down = jnp.concatenate([cur[:, W:], down_row], axis=1)
    colsum = up + cur + down

    wpos = jax.lax.broadcasted_iota(jnp.int32, (1, NB), 1) % W
    left = jnp.concatenate([colsum[:, :1], colsum[:, :-1]], axis=1)
    left = jnp.where(wpos == 0, colsum, left)
    right = jnp.concatenate([colsum[:, 1:], colsum[:, -1:]], axis=1)
    right = jnp.where(wpos == W - 1, colsum, right)
    sum9 = left + colsum + right

    mean = (sum9 - cur) * 0.125

    agg = jnp.concatenate([cur, mean], axis=0)
    y = jnp.dot(w_ref[...], agg, preferred_element_type=jnp.float32)
    y = y + b_ref[...]
    out_ref[0] = jnp.maximum(y, 0.0) + cur


def kernel(x, W_proj, b_proj):
    B, C, H, W = x.shape
    N = H * W
    HB = 28
    nbands = H // HB
    NB = HB * W
    RPB = HB // 4

    x2 = x.reshape(B, C, N)
    b2 = b_proj.reshape(C, 1)

    grid = (B, nbands)
    out = pl.pallas_call(
        functools.partial(_band_kernel, W=W, HB=HB),
        grid=grid,
        in_specs=[
            pl.BlockSpec((1, C, NB), lambda b, h: (b, 0, h)),
            pl.BlockSpec(
                (1, C, 4 * W),
                lambda b, h: (b, 0, jnp.maximum(h * RPB - 1, 0)),
            ),
            pl.BlockSpec(
                (1, C, 4 * W),
                lambda b, h: (b, 0, jnp.minimum((h + 1) * RPB, nbands * RPB - 1)),
            ),
            pl.BlockSpec((C, 2 * C), lambda b, h: (0, 0)),
            pl.BlockSpec((C, 1), lambda b, h: (0, 0)),
        ],
        out_specs=pl.BlockSpec((1, C, NB), lambda b, h: (b, 0, h)),
        out_shape=jax.ShapeDtypeStruct((B, C, N), jnp.float32),
        compiler_params=pltpu.CompilerParams(
            dimension_semantics=("parallel", "arbitrary"),
        ),
    )(x2, x2, x2, W_proj, b2)
    return out.reshape(B, C, H, W)
